# baseline (speedup 1.0000x reference)
import functools

import jax
import jax.numpy as jnp
from jax import lax
from jax.experimental import pallas as pl
from jax.experimental.pallas import tpu as pltpu


def _pool2x2(ref, H, W):
    a = jnp.maximum(ref[0::2, :], ref[1::2, :])
    a4 = a.reshape(H // 2, 2, W // 2, a.shape[-1])
    c = jnp.max(a4, axis=1)
    return c.reshape(-1, a.shape[-1])


def _fused_kernel(x_ref, phiw_ref, gw_ref, thetaw_ref, ww_ref, thetab_ref,
                  phib_ref, gb_ref, wb_ref, gamma_ref, beta_ref, mean_ref,
                  var_ref, o_ref, *scratch, G, H, W, D, inv_nkv):
    N = H * W
    Nkv = N // 4
    phi_refs = scratch[:G]
    g_refs = scratch[G:2 * G]
    wpg_ref, phiall_ref, gall_ref, q_ref, r_ref = scratch[2 * G:]

    @pl.when(pl.program_id(0) == 0)
    def _():
        wpg_ref[:, :D] = phiw_ref[...]
        wpg_ref[:, D:] = gw_ref[...]

    scale = gamma_ref[...] * lax.rsqrt(var_ref[...] + 1e-5)
    wfold = ww_ref[...] * (scale * inv_nkv)
    bfold = (wb_ref[...] - mean_ref[...]) * scale + beta_ref[...]
    wpg = wpg_ref[...]
    wtheta = thetaw_ref[...]
    btheta = thetab_ref[...]
    bphi = phib_ref[...]
    bg = gb_ref[...]

    for gi in range(G):
        pg = jnp.dot(x_ref[gi], wpg,
                     preferred_element_type=jnp.float32)
        phi_refs[gi][...] = pg[:, :D]
        g_refs[gi][...] = pg[:, D:]
        phiall_ref[gi * Nkv:(gi + 1) * Nkv, :] = (
            _pool2x2(phi_refs[gi], H, W) + bphi)
        gall_ref[gi * Nkv:(gi + 1) * Nkv, :] = (
            _pool2x2(g_refs[gi], H, W) + bg)

    q_ref[...] = lax.dot_general(
        phiall_ref[...], wtheta, (((1,), (1,)), ((), ())),
        preferred_element_type=jnp.float32)
    r_ref[...] = jnp.dot(gall_ref[...], wfold,
                         preferred_element_type=jnp.float32)
    s = lax.dot_general(
        phiall_ref[...], btheta, (((1,), (1,)), ((), ())),
        preferred_element_type=jnp.float32)

    for gi in range(G):
        qg = q_ref[gi * Nkv:(gi + 1) * Nkv, :]
        rg = r_ref[gi * Nkv:(gi + 1) * Nkv, :]
        wc = lax.dot_general(
            qg, rg, (((0,), (0,)), ((), ())),
            preferred_element_type=jnp.float32)
        bc = lax.dot_general(
            s[gi * Nkv:(gi + 1) * Nkv, :], rg, (((0,), (0,)), ((), ())),
            preferred_element_type=jnp.float32) + bfold
        x = x_ref[gi]
        o_ref[gi] = (
            jnp.dot(x, wc, preferred_element_type=jnp.float32)
            + bc + x
        ).astype(o_ref.dtype)


@jax.jit
def kernel(x, theta_w, theta_b, phi_w, phi_b, g_w, g_b, W_w, W_b,
           bn_gamma, bn_beta, bn_mean, bn_var):
    B, C, H, W = x.shape
    D = theta_w.shape[1]
    N = H * W
    Nkv = (H // 2) * (W // 2)
    G = 8 if B % 8 == 0 else (4 if B % 4 == 0 else (2 if B % 2 == 0 else 1))

    x_tok = jnp.transpose(x, (0, 2, 3, 1)).reshape(B, N, C)

    row = lambda v: v[None, :]
    full = lambda r, c: pl.BlockSpec((r, c), lambda b: (0, 0))
    z = pl.pallas_call(
        functools.partial(_fused_kernel, G=G, H=H, W=W, D=D,
                          inv_nkv=1.0 / Nkv),
        out_shape=jax.ShapeDtypeStruct((B, N, C), x.dtype),
        grid=(B // G,),
        in_specs=[
            pl.BlockSpec((G, N, C), lambda b: (b, 0, 0)),
            full(C, D), full(C, D), full(C, D), full(D, C),
            full(1, D), full(1, D), full(1, D),
            full(1, C), full(1, C), full(1, C), full(1, C), full(1, C),
        ],
        out_specs=pl.BlockSpec((G, N, C), lambda b: (b, 0, 0)),
        scratch_shapes=(
            [pltpu.VMEM((N, D), jnp.float32) for _ in range(2 * G)]
            + [pltpu.VMEM((C, 2 * D), jnp.float32),
               pltpu.VMEM((G * Nkv, D), jnp.float32),
               pltpu.VMEM((G * Nkv, D), jnp.float32),
               pltpu.VMEM((G * Nkv, C), jnp.float32),
               pltpu.VMEM((G * Nkv, C), jnp.float32)]),
        compiler_params=pltpu.CompilerParams(
            dimension_semantics=("parallel",)),
    )(x_tok, phi_w, g_w, theta_w, W_w, row(theta_b), row(phi_b), row(g_b),
      row(W_b), row(bn_gamma), row(bn_beta), row(bn_mean), row(bn_var))
    return jnp.transpose(z.reshape(B, H, W, C), (0, 3, 1, 2))


# speedup vs baseline: 1.0043x; 1.0043x over previous
import functools

import jax
import jax.numpy as jnp
from jax import lax
from jax.experimental import pallas as pl
from jax.experimental.pallas import tpu as pltpu


def _pool2x2(ref, H, W):
    a = jnp.maximum(ref[0::2, :], ref[1::2, :])
    a4 = a.reshape(H // 2, 2, W // 2, a.shape[-1])
    c = jnp.max(a4, axis=1)
    return c.reshape(-1, a.shape[-1])


def _fused_kernel(x_ref, phiw_ref, gw_ref, thetaw_ref, ww_ref, thetab_ref,
                  phib_ref, gb_ref, wb_ref, gamma_ref, beta_ref, mean_ref,
                  var_ref, o_ref, *scratch, G, H, W, D, inv_nkv):
    N = H * W
    Nkv = N // 4
    phi_refs = scratch[:G]
    g_refs = scratch[G:2 * G]
    wpg_ref, phiall_ref, gall_ref, q_ref, r_ref = scratch[2 * G:]

    wpg_ref[:, :D] = phiw_ref[...]
    wpg_ref[:, D:] = gw_ref[...]

    scale = gamma_ref[...] * lax.rsqrt(var_ref[...] + 1e-5)
    wfold = ww_ref[...] * (scale * inv_nkv)
    bfold = (wb_ref[...] - mean_ref[...]) * scale + beta_ref[...]
    wpg = wpg_ref[...]
    wtheta = thetaw_ref[...]
    btheta = thetab_ref[...]
    bphi = phib_ref[...]
    bg = gb_ref[...]

    for gi in range(G):
        pg = jnp.dot(x_ref[gi], wpg,
                     preferred_element_type=jnp.float32)
        phi_refs[gi][...] = pg[:, :D]
        g_refs[gi][...] = pg[:, D:]
        phiall_ref[gi * Nkv:(gi + 1) * Nkv, :] = (
            _pool2x2(phi_refs[gi], H, W) + bphi)
        gall_ref[gi * Nkv:(gi + 1) * Nkv, :] = (
            _pool2x2(g_refs[gi], H, W) + bg)

    q_ref[...] = lax.dot_general(
        phiall_ref[...], wtheta, (((1,), (1,)), ((), ())),
        preferred_element_type=jnp.float32)
    r_ref[...] = jnp.dot(gall_ref[...], wfold,
                         preferred_element_type=jnp.float32)
    s = lax.dot_general(
        phiall_ref[...], btheta, (((1,), (1,)), ((), ())),
        preferred_element_type=jnp.float32)

    for gi in range(G):
        qg = q_ref[gi * Nkv:(gi + 1) * Nkv, :]
        rg = r_ref[gi * Nkv:(gi + 1) * Nkv, :]
        wc = lax.dot_general(
            qg, rg, (((0,), (0,)), ((), ())),
            preferred_element_type=jnp.float32)
        bc = lax.dot_general(
            s[gi * Nkv:(gi + 1) * Nkv, :], rg, (((0,), (0,)), ((), ())),
            preferred_element_type=jnp.float32) + bfold
        x = x_ref[gi]
        o_ref[gi] = (
            jnp.dot(x, wc, preferred_element_type=jnp.float32)
            + bc + x
        ).astype(o_ref.dtype)


@jax.jit
def kernel(x, theta_w, theta_b, phi_w, phi_b, g_w, g_b, W_w, W_b,
           bn_gamma, bn_beta, bn_mean, bn_var):
    B, C, H, W = x.shape
    D = theta_w.shape[1]
    N = H * W
    Nkv = (H // 2) * (W // 2)
    G = 8 if B % 8 == 0 else (4 if B % 4 == 0 else (2 if B % 2 == 0 else 1))

    x_tok = jnp.transpose(x, (0, 2, 3, 1)).reshape(B, N, C)

    row = lambda v: v[None, :]
    full = lambda r, c: pl.BlockSpec((r, c), lambda b: (0, 0))
    z = pl.pallas_call(
        functools.partial(_fused_kernel, G=G, H=H, W=W, D=D,
                          inv_nkv=1.0 / Nkv),
        out_shape=jax.ShapeDtypeStruct((B, N, C), x.dtype),
        grid=(B // G,),
        in_specs=[
            pl.BlockSpec((G, N, C), lambda b: (b, 0, 0)),
            full(C, D), full(C, D), full(C, D), full(D, C),
            full(1, D), full(1, D), full(1, D),
            full(1, C), full(1, C), full(1, C), full(1, C), full(1, C),
        ],
        out_specs=pl.BlockSpec((G, N, C), lambda b: (b, 0, 0)),
        scratch_shapes=(
            [pltpu.VMEM((N, D), jnp.float32) for _ in range(2 * G)]
            + [pltpu.VMEM((C, 2 * D), jnp.float32),
               pltpu.VMEM((G * Nkv, D), jnp.float32),
               pltpu.VMEM((G * Nkv, D), jnp.float32),
               pltpu.VMEM((G * Nkv, C), jnp.float32),
               pltpu.VMEM((G * Nkv, C), jnp.float32)]),
        compiler_params=pltpu.CompilerParams(
            dimension_semantics=("parallel",)),
    )(x_tok, phi_w, g_w, theta_w, W_w, row(theta_b), row(phi_b), row(g_b),
      row(W_b), row(bn_gamma), row(bn_beta), row(bn_mean), row(bn_var))
    return jnp.transpose(z.reshape(B, H, W, C), (0, 3, 1, 2))
